# Initial kernel scaffold; baseline (speedup 1.0000x reference)
import jax
import jax.numpy as jnp
from jax.experimental import pallas as pl


def kernel(hidden_states, gate_w, gate_up_w, down_w):
    raise NotImplementedError("write your pallas kernel here")


# speedup vs baseline: 7.7115x; 7.7115x over previous
import functools

import jax
import jax.numpy as jnp
from jax.experimental import pallas as pl

NUM_EXPERTS = 8
TOP_K = 2
MODEL_DIM = 768
INTER_DIM = 768


def _moe_body(x_ref, gwt_ref, u_ref, d_ref, out_ref):
    x = x_ref[...]
    gwt = gwt_ref[...]
    logits = jax.lax.dot_general(
        x, gwt, (((1,), (0,)), ((), ())),
        preferred_element_type=jnp.float32,
    )

    idx = jax.lax.broadcasted_iota(jnp.int32, logits.shape, 1)
    r1 = jnp.max(logits, axis=1, keepdims=True)
    i1 = jnp.min(jnp.where(logits == r1, idx, NUM_EXPERTS),
                 axis=1, keepdims=True)
    mask1 = idx == i1
    l2 = jnp.where(mask1, -jnp.inf, logits)
    r2 = jnp.max(l2, axis=1, keepdims=True)
    i2 = jnp.min(jnp.where(l2 == r2, idx, NUM_EXPERTS),
                 axis=1, keepdims=True)
    sel = (mask1 | (idx == i2)).astype(jnp.float32)

    xb = x.astype(jnp.bfloat16)
    acc = jnp.zeros(out_ref.shape, dtype=jnp.float32)
    for e in range(NUM_EXPERTS):
        u = jax.lax.dot_general(
            xb, u_ref[e], (((1,), (0,)), ((), ())),
            preferred_element_type=jnp.float32)
        h = (u * u) * sel[:, e:e + 1]
        acc = acc + jax.lax.dot_general(
            h.astype(jnp.bfloat16), d_ref[e], (((1,), (0,)), ((), ())),
            preferred_element_type=jnp.float32)
    out_ref[...] = acc


@functools.partial(jax.jit, static_argnames=("interpret",))
def kernel(hidden_states, gate_w, gate_up_w, down_w, interpret=False):
    orig_shape = hidden_states.shape
    x = hidden_states.reshape(-1, orig_shape[-1])
    t = x.shape[0]
    bt = 256
    u = gate_up_w[:, :, INTER_DIM:].astype(jnp.bfloat16)
    d = down_w.astype(jnp.bfloat16)
    gwt = gate_w.T

    out = pl.pallas_call(
        _moe_body,
        grid=(t // bt,),
        in_specs=[
            pl.BlockSpec((bt, MODEL_DIM), lambda i: (i, 0)),
            pl.BlockSpec((MODEL_DIM, NUM_EXPERTS), lambda i: (0, 0)),
            pl.BlockSpec((NUM_EXPERTS, MODEL_DIM, INTER_DIM), lambda i: (0, 0, 0)),
            pl.BlockSpec((NUM_EXPERTS, INTER_DIM, MODEL_DIM), lambda i: (0, 0, 0)),
        ],
        out_specs=pl.BlockSpec((bt, MODEL_DIM), lambda i: (i, 0)),
        out_shape=jax.ShapeDtypeStruct((t, MODEL_DIM), jnp.float32),
        interpret=interpret,
    )(x, gwt, u, d)
    return out.reshape(orig_shape)
